# baseline (speedup 1.0000x reference)
import functools

import jax
import jax.numpy as jnp
from jax import lax
from jax.experimental import pallas as pl
from jax.experimental.pallas import tpu as pltpu
from jax.experimental.pallas import tpu_sc as plsc

B = 16384
NNUM = 13
NF = 26
V = 100000
D = 16
IN = NNUM + NF * D
H1 = 128
H2 = 64

NC, NS = 2, 16
NW = NC * NS
ROWS = B * NF
RPW = ROWS // NW
CH = 6656
NCHUNK = RPW // CH

@functools.cache
def _make_sc_gather(rows):
    mesh = plsc.VectorSubcoreMesh(
        core_axis_name="c", subcore_axis_name="s", num_cores=NC, num_subcores=NS
    )
    rpw = rows // NW
    nchunk = (rpw + CH - 1) // CH

    @functools.partial(
        pl.kernel,
        out_type=jax.ShapeDtypeStruct((rows, D), jnp.float32),
        mesh=mesh,
        scratch_types=[
            pltpu.VMEM((CH,), jnp.int32),
            pltpu.VMEM((CH, D), jnp.float32),
            pltpu.SemaphoreType.DMA,
        ],
        compiler_params=pltpu.CompilerParams(use_tc_tiling_on_sc=False),
    )
    def _sc_gather(idx_hbm, table_hbm, out_hbm, idx_v, rows_v, sem):
        wid = lax.axis_index("s") * NC + lax.axis_index("c")
        base = wid * rpw
        for i in range(nchunk):
            off = base + i * CH
            pltpu.sync_copy(idx_hbm.at[pl.ds(off, CH)], idx_v)
            pltpu.async_copy(table_hbm.at[idx_v], rows_v, sem).wait()
            pltpu.sync_copy(rows_v, out_hbm.at[pl.ds(off, CH)])

    return _sc_gather


NPLANE = 4
VB = 5120


def _tr_body(et_ref, out_ref):
    y = jnp.transpose(et_ref[...], (1, 0))
    for k in range(NPLANE):
        lo = k * 128
        hi = min((k + 1) * 128, NF * D)
        out_ref[k, :, : hi - lo] = y[:, lo:hi]


_transpose = pl.pallas_call(
    _tr_body,
    grid=((V + VB - 1) // VB,),
    in_specs=[pl.BlockSpec((NF * D, VB), lambda j: (0, j))],
    out_specs=pl.BlockSpec((NPLANE, VB, 128), lambda j: (0, j, 0)),
    out_shape=jax.ShapeDtypeStruct((NPLANE, V, 128), jnp.float32),
)


BLK = 2048


def _mlp_body(xn_ref, xe_ref, w1n_ref, w1e_ref, b1_ref, w2_ref, b2_ref,
              w3_ref, b3_ref, out_ref):
    h1 = jnp.dot(xe_ref[...], w1e_ref[...], preferred_element_type=jnp.float32)
    h1 += jnp.dot(xn_ref[...], w1n_ref[...], preferred_element_type=jnp.float32)
    h1 = jnp.maximum(h1 + b1_ref[...], 0.0)
    h2 = jnp.maximum(
        jnp.dot(h1, w2_ref[...], preferred_element_type=jnp.float32) + b2_ref[...],
        0.0,
    )
    out_ref[...] = (
        jnp.dot(h2, w3_ref[...], preferred_element_type=jnp.float32) + b3_ref[...]
    )


@functools.cache
def _make_mlp(nb):
    return pl.pallas_call(
        _mlp_body,
        grid=(nb // BLK,),
        in_specs=[
            pl.BlockSpec((BLK, NNUM), lambda i: (i, 0)),
            pl.BlockSpec((BLK, NF * D), lambda i: (i, 0)),
            pl.BlockSpec((NNUM, H1), lambda i: (0, 0)),
            pl.BlockSpec((NF * D, H1), lambda i: (0, 0)),
            pl.BlockSpec((1, H1), lambda i: (0, 0)),
            pl.BlockSpec((H1, H2), lambda i: (0, 0)),
            pl.BlockSpec((1, H2), lambda i: (0, 0)),
            pl.BlockSpec((H2, 1), lambda i: (0, 0)),
            pl.BlockSpec((1, 1), lambda i: (0, 0)),
        ],
        out_specs=pl.BlockSpec((BLK, 1), lambda i: (i, 0)),
        out_shape=jax.ShapeDtypeStruct((nb, 1), jnp.float32),
    )


def kernel(x_num, x_cat, E, W1, b1, W2, b2, W3, b3):
    f = jnp.arange(NF, dtype=jnp.int32)
    off = (f // 8) * (8 * V) + f % 8
    idx = (x_cat * 8 + off[None, :]).reshape(ROWS)
    table = _transpose(
        jnp.transpose(E, (0, 2, 1)).reshape(NF * D, V)
    ).reshape(NPLANE * V * 8, D)
    nb = B // 2
    halves = []
    for h in range(2):
        emb = _make_sc_gather(nb * NF)(idx[h * nb * NF:(h + 1) * nb * NF], table)
        x_emb = emb.reshape(nb, NF * D)
        halves.append(_make_mlp(nb)(
            x_num[h * nb:(h + 1) * nb], x_emb,
            W1[:NNUM], W1[NNUM:], b1[None, :],
            W2, b2[None, :],
            W3, b3[None, :],
        ))
    return jnp.concatenate(halves, axis=0)


# speedup vs baseline: 1.0046x; 1.0046x over previous
import functools

import jax
import jax.numpy as jnp
from jax import lax
from jax.experimental import pallas as pl
from jax.experimental.pallas import tpu as pltpu
from jax.experimental.pallas import tpu_sc as plsc

B = 16384
NNUM = 13
NF = 26
V = 100000
D = 16
IN = NNUM + NF * D
H1 = 128
H2 = 64

NC, NS = 2, 16
NW = NC * NS
ROWS = B * NF
RPW = ROWS // NW
CH = 3328
NCHUNK = RPW // CH

@functools.cache
def _make_sc_gather(rows):
    mesh = plsc.VectorSubcoreMesh(
        core_axis_name="c", subcore_axis_name="s", num_cores=NC, num_subcores=NS
    )
    rpw = rows // NW
    nchunk = (rpw + CH - 1) // CH

    @functools.partial(
        pl.kernel,
        out_type=jax.ShapeDtypeStruct((rows, D), jnp.float32),
        mesh=mesh,
        scratch_types=[
            pltpu.VMEM((CH,), jnp.int32),
            pltpu.VMEM((CH, D), jnp.float32),
            pltpu.SemaphoreType.DMA,
        ],
        compiler_params=pltpu.CompilerParams(use_tc_tiling_on_sc=False),
    )
    def _sc_gather(idx_hbm, table_hbm, out_hbm, idx_v, rows_v, sem):
        wid = lax.axis_index("s") * NC + lax.axis_index("c")
        base = wid * rpw
        for i in range(nchunk):
            off = base + i * CH
            pltpu.sync_copy(idx_hbm.at[pl.ds(off, CH)], idx_v)
            pltpu.async_copy(table_hbm.at[idx_v], rows_v, sem).wait()
            pltpu.sync_copy(rows_v, out_hbm.at[pl.ds(off, CH)])

    return _sc_gather


NPLANE = 4
VB = 5120


def _tr_body(et_ref, out_ref):
    y = jnp.transpose(et_ref[...], (1, 0))
    for k in range(NPLANE):
        lo = k * 128
        hi = min((k + 1) * 128, NF * D)
        out_ref[k, :, : hi - lo] = y[:, lo:hi]


_transpose = pl.pallas_call(
    _tr_body,
    grid=((V + VB - 1) // VB,),
    in_specs=[pl.BlockSpec((NF * D, VB), lambda j: (0, j))],
    out_specs=pl.BlockSpec((NPLANE, VB, 128), lambda j: (0, j, 0)),
    out_shape=jax.ShapeDtypeStruct((NPLANE, V, 128), jnp.float32),
)


BLK = 2048


def _mlp_body(xn_ref, xe_ref, w1n_ref, w1e_ref, b1_ref, w2_ref, b2_ref,
              w3_ref, b3_ref, out_ref):
    h1 = jnp.dot(xe_ref[...], w1e_ref[...], preferred_element_type=jnp.float32)
    h1 += jnp.dot(xn_ref[...], w1n_ref[...], preferred_element_type=jnp.float32)
    h1 = jnp.maximum(h1 + b1_ref[...], 0.0)
    h2 = jnp.maximum(
        jnp.dot(h1, w2_ref[...], preferred_element_type=jnp.float32) + b2_ref[...],
        0.0,
    )
    out_ref[...] = (
        jnp.dot(h2, w3_ref[...], preferred_element_type=jnp.float32) + b3_ref[...]
    )


@functools.cache
def _make_mlp(nb):
    return pl.pallas_call(
        _mlp_body,
        grid=(nb // BLK,),
        in_specs=[
            pl.BlockSpec((BLK, NNUM), lambda i: (i, 0)),
            pl.BlockSpec((BLK, NF * D), lambda i: (i, 0)),
            pl.BlockSpec((NNUM, H1), lambda i: (0, 0)),
            pl.BlockSpec((NF * D, H1), lambda i: (0, 0)),
            pl.BlockSpec((1, H1), lambda i: (0, 0)),
            pl.BlockSpec((H1, H2), lambda i: (0, 0)),
            pl.BlockSpec((1, H2), lambda i: (0, 0)),
            pl.BlockSpec((H2, 1), lambda i: (0, 0)),
            pl.BlockSpec((1, 1), lambda i: (0, 0)),
        ],
        out_specs=pl.BlockSpec((BLK, 1), lambda i: (i, 0)),
        out_shape=jax.ShapeDtypeStruct((nb, 1), jnp.float32),
    )


def kernel(x_num, x_cat, E, W1, b1, W2, b2, W3, b3):
    f = jnp.arange(NF, dtype=jnp.int32)
    off = (f // 8) * (8 * V) + f % 8
    idx = (x_cat * 8 + off[None, :]).reshape(ROWS)
    table = _transpose(
        jnp.transpose(E, (0, 2, 1)).reshape(NF * D, V)
    ).reshape(NPLANE * V * 8, D)
    nb = B // 2
    halves = []
    for h in range(2):
        emb = _make_sc_gather(nb * NF)(idx[h * nb * NF:(h + 1) * nb * NF], table)
        x_emb = emb.reshape(nb, NF * D)
        halves.append(_make_mlp(nb)(
            x_num[h * nb:(h + 1) * nb], x_emb,
            W1[:NNUM], W1[NNUM:], b1[None, :],
            W2, b2[None, :],
            W3, b3[None, :],
        ))
    return jnp.concatenate(halves, axis=0)
